# baseline (speedup 1.0000x reference)
import functools

import jax
import jax.numpy as jnp
from jax import lax
from jax.experimental import pallas as pl
from jax.experimental.pallas import tpu as pltpu
from jax.experimental.pallas import tpu_sc as plsc

B, NN, F, V, D = 16384, 13, 26, 100000, 32
NC, NS = 2, 16
NW = NC * NS
RPW = B // NW
GCH = 128
NCH = RPW // GCH
T = 1 + NN + F
H = D // 16


def _sc_body(tab, xcat, xnum, wts, bias, cls, out,
             idxb, xcb, rows, numbuf, xnb, wbuf, bbuf, clsb, sem):
    wid = lax.axis_index("s") * NC + lax.axis_index("c")
    base = wid * RPW

    pltpu.sync_copy(xcat.at[pl.ds(base, RPW)], xcb)
    basev = lax.iota(jnp.int32, 16)

    def f_body(f, carry):
        fv16 = jnp.full((16,), f, jnp.int32)
        for j in range(RPW // 16):
            iv = basev + (j * 16)
            idxb[pl.ds(j * 16, 16)] = plsc.load_gather(xcb, [iv, fv16]) * 2
        pltpu.async_copy(tab.at[f].at[idxb], rows, sem).wait()
        pltpu.sync_copy(rows, out.at[pl.ds(base, RPW), 1 + NN + f, pl.ds(0, D)])
        return carry

    lax.fori_loop(0, F, f_body, 0)

    pltpu.sync_copy(wts, wbuf)
    pltpu.sync_copy(bias, bbuf)
    pltpu.sync_copy(cls, clsb)

    for c in range(NCH):
        pltpu.sync_copy(xnum.at[pl.ds(base + c * GCH, GCH)], xnb)

        cv = [clsb[0, pl.ds(h * 16, 16)] for h in range(H)]

        def cls_iter(i, carry, cv=cv):
            for h in range(H):
                numbuf[i, 0, pl.ds(h * 16, 16)] = cv[h]
            return carry

        lax.fori_loop(0, GCH, cls_iter, 0)

        for n in range(NN):
            wv = [wbuf[n, pl.ds(h * 16, 16)] for h in range(H)]
            bv = [bbuf[n, pl.ds(h * 16, 16)] for h in range(H)]
            nv = jnp.full((16,), n, jnp.int32)

            def num_iter(i, carry, wv=wv, bv=bv, nv=nv, n=n):
                iv = jnp.full((16,), i, jnp.int32)
                sv = plsc.load_gather(xnb, [iv, nv])
                for h in range(H):
                    numbuf[i, 1 + n, pl.ds(h * 16, 16)] = sv * wv[h] + bv[h]
                return carry

            lax.fori_loop(0, GCH, num_iter, 0)

        pltpu.sync_copy(numbuf, out.at[pl.ds(base + c * GCH, GCH), pl.ds(0, 1 + NN),
                                       pl.ds(0, D)])


@functools.cache
def _sc_call():
    mesh = plsc.VectorSubcoreMesh(core_axis_name="c", subcore_axis_name="s")
    return pl.kernel(
        _sc_body,
        mesh=mesh,
        compiler_params=pltpu.CompilerParams(use_tc_tiling_on_sc=False,
                                             needs_layout_passes=False),
        out_type=jax.ShapeDtypeStruct((B, T, 128), jnp.float32),
        scratch_types=[
            pltpu.VMEM((RPW,), jnp.int32),
            pltpu.VMEM((RPW, F), jnp.int32),
            pltpu.VMEM((RPW, D), jnp.float32),
            pltpu.VMEM((GCH, 1 + NN, D), jnp.float32),
            pltpu.VMEM((GCH, NN), jnp.float32),
            pltpu.VMEM((NN, D), jnp.float32),
            pltpu.VMEM((NN, D), jnp.float32),
            pltpu.VMEM((1, D), jnp.float32),
            pltpu.SemaphoreType.DMA,
        ],
    )


@jax.jit
def _impl(x_num, x_cat, num_weights, num_bias, cat_tables, cls_token):
    cls = cls_token.reshape(1, D)
    tabp = jnp.pad(cat_tables, ((0, 0), (0, 0), (0, 64 - D))).reshape(F, V * 2, D)
    outp = _sc_call()(tabp, x_cat, x_num, num_weights, num_bias, cls)
    return outp[:, :, :D]


def kernel(x_num, x_cat, num_weights, num_bias, cat_tables, cls_token):
    return _impl(x_num, x_cat, num_weights, num_bias, cat_tables, cls_token)


# speedup vs baseline: 1.7182x; 1.7182x over previous
import functools

import jax
import jax.numpy as jnp
from jax import lax
from jax.experimental import pallas as pl
from jax.experimental.pallas import tpu as pltpu
from jax.experimental.pallas import tpu_sc as plsc

B, NN, F, V, D = 16384, 13, 26, 100000, 32
NC, NS = 2, 16
NW = NC * NS
RPW = B // NW
GCH = 128
NCH = RPW // GCH
T = 1 + NN + F
H = D // 16


def _sc_body(tab, xcat, xnum, wts, bias, cls, out,
             idxb, xcb, rows, numbuf, xnb, wbuf, bbuf, clsb, sem):
    wid = lax.axis_index("s") * NC + lax.axis_index("c")
    base = wid * RPW

    pltpu.sync_copy(xcat.at[pl.ds(base, RPW)], xcb)
    basev = lax.iota(jnp.int32, 16)

    def f_body(f, carry):
        fv16 = jnp.full((16,), f, jnp.int32)
        for j in range(RPW // 16):
            iv = basev + (j * 16)
            idxb[pl.ds(j * 16, 16)] = plsc.load_gather(xcb, [iv, fv16]) * 4
        pltpu.async_copy(tab.at[f].at[idxb], rows, sem).wait()
        pltpu.sync_copy(rows, out.at[pl.ds(base, RPW), 1 + NN + f, pl.ds(0, D)])
        return carry

    lax.fori_loop(0, F, f_body, 0)

    pltpu.sync_copy(wts, wbuf)
    pltpu.sync_copy(bias, bbuf)
    pltpu.sync_copy(cls, clsb)

    for c in range(NCH):
        pltpu.sync_copy(xnum.at[pl.ds(base + c * GCH, GCH)], xnb)

        cv = [clsb[0, pl.ds(h * 16, 16)] for h in range(H)]

        def cls_iter(i, carry, cv=cv):
            for h in range(H):
                numbuf[i, 0, pl.ds(h * 16, 16)] = cv[h]
            return carry

        lax.fori_loop(0, GCH, cls_iter, 0)

        for n in range(NN):
            wv = [wbuf[n, pl.ds(h * 16, 16)] for h in range(H)]
            bv = [bbuf[n, pl.ds(h * 16, 16)] for h in range(H)]
            nv = jnp.full((16,), n, jnp.int32)

            def num_iter(i, carry, wv=wv, bv=bv, nv=nv, n=n):
                iv = jnp.full((16,), i, jnp.int32)
                sv = plsc.load_gather(xnb, [iv, nv])
                for h in range(H):
                    numbuf[i, 1 + n, pl.ds(h * 16, 16)] = sv * wv[h] + bv[h]
                return carry

            lax.fori_loop(0, GCH, num_iter, 0)

        pltpu.sync_copy(numbuf, out.at[pl.ds(base + c * GCH, GCH), pl.ds(0, 1 + NN),
                                       pl.ds(0, D)])


@functools.cache
def _sc_call():
    mesh = plsc.VectorSubcoreMesh(core_axis_name="c", subcore_axis_name="s")
    return pl.kernel(
        _sc_body,
        mesh=mesh,
        compiler_params=pltpu.CompilerParams(use_tc_tiling_on_sc=False,
                                             needs_layout_passes=False),
        out_type=jax.ShapeDtypeStruct((B, T, 128), jnp.float32),
        scratch_types=[
            pltpu.VMEM((RPW,), jnp.int32),
            pltpu.VMEM((RPW, F), jnp.int32),
            pltpu.VMEM((RPW, D), jnp.float32),
            pltpu.VMEM((GCH, 1 + NN, D), jnp.float32),
            pltpu.VMEM((GCH, NN), jnp.float32),
            pltpu.VMEM((NN, D), jnp.float32),
            pltpu.VMEM((NN, D), jnp.float32),
            pltpu.VMEM((1, D), jnp.float32),
            pltpu.SemaphoreType.DMA,
        ],
    )


@jax.jit
def _impl(x_num, x_cat, num_weights, num_bias, cat_tables, cls_token):
    cls = cls_token.reshape(1, D)
    tabp = jnp.pad(cat_tables, ((0, 0), (0, 0), (0, 128 - D))).reshape(F, V * 4, D)
    outp = _sc_call()(tabp, x_cat, x_num, num_weights, num_bias, cls)
    return outp[:, :, :D]


def kernel(x_num, x_cat, num_weights, num_bias, cat_tables, cls_token):
    return _impl(x_num, x_cat, num_weights, num_bias, cat_tables, cls_token)
